# baseline (speedup 1.0000x reference)
import jax
import jax.numpy as jnp
from jax.experimental import pallas as pl
from jax.experimental.pallas import tpu as pltpu

B, T, C, D = 256, 128, 141, 1536
H = 141
G = 4 * H
L = 4
TP = 144
TC = 16
NCH = TP // TC
SP = 256
GP = 4 * SP
BB = 8
BK = B // 2

_f32 = jnp.float32


def _cparams(sems):
    return pltpu.CompilerParams(dimension_semantics=sems)


def _k1_body(wlinT_ref, wcatT_ref, o_ref):
    o_ref[...] = jnp.dot(wlinT_ref[...], wcatT_ref[...],
                         preferred_element_type=_f32).astype(jnp.bfloat16)


def _k1(wlinT, wcatT):
    return pl.pallas_call(
        _k1_body,
        grid=(1,),
        in_specs=[
            pl.BlockSpec((D, D), lambda i: (0, 0)),
            pl.BlockSpec((D, 2 * G), lambda i: (0, 0)),
        ],
        out_specs=pl.BlockSpec((D, 2 * G), lambda i: (0, 0)),
        out_shape=jax.ShapeDtypeStruct((D, 2 * G), jnp.bfloat16),
        compiler_params=_cparams(("arbitrary",)),
    )(wlinT, wcatT)


def _k2_body(x_ref, wc_ref, blin_ref, st_ref, en_ref, cgf_ref, cgb_ref,
             hi_ref, lo_ref):
    xg = jnp.dot(x_ref[...].reshape(BB * T, D).astype(jnp.bfloat16),
                 wc_ref[...],
                 preferred_element_type=_f32) + blin_ref[...]
    bits = jax.lax.bitcast_convert_type(xg, jnp.int32)
    hi = jax.lax.bitcast_convert_type(
        jnp.bitwise_and(bits, jnp.int32(-65536)), _f32)
    hi_ref[...] = hi.astype(jnp.bfloat16)
    lo_ref[...] = (xg - hi).astype(jnp.bfloat16)
    c_col = jax.lax.broadcasted_iota(jnp.int32, (TP, T), 0)
    for r in range(BB):
        st = st_ref[r, :].reshape(1, T)
        en = en_ref[r, :].reshape(1, T)
        cover = ((st <= c_col) & (c_col < en)).astype(jnp.bfloat16)
        row = (jnp.dot(cover, hi_ref[r * T:(r + 1) * T, :],
                       preferred_element_type=_f32)
               + jnp.dot(cover, lo_ref[r * T:(r + 1) * T, :],
                         preferred_element_type=_f32))
        for k in range(4):
            cgf_ref[k, :, r, :] = row[:, k * H:(k + 1) * H]
            cgb_ref[k, :, r, :] = row[:, G + k * H:G + (k + 1) * H]


def _k2(tok_feats, wcomb, blin, starts, ends):
    nb = B // BB
    return pl.pallas_call(
        _k2_body,
        grid=(2, nb // 2),
        in_specs=[
            pl.BlockSpec((BB, T, D), lambda i, j: (i * (nb // 2) + j, 0, 0)),
            pl.BlockSpec((D, 2 * G), lambda i, j: (0, 0)),
            pl.BlockSpec((1, 2 * G), lambda i, j: (0, 0)),
            pl.BlockSpec((BB, T), lambda i, j: (i * (nb // 2) + j, 0)),
            pl.BlockSpec((BB, T), lambda i, j: (i * (nb // 2) + j, 0)),
        ],
        out_specs=[
            pl.BlockSpec((4, TP, BB, H),
                         lambda i, j: (0, 0, i * (nb // 2) + j, 0)),
            pl.BlockSpec((4, TP, BB, H),
                         lambda i, j: (0, 0, i * (nb // 2) + j, 0)),
        ],
        out_shape=[
            jax.ShapeDtypeStruct((4, TP, B, H), _f32),
            jax.ShapeDtypeStruct((4, TP, B, H), _f32),
        ],
        scratch_shapes=[pltpu.VMEM((BB * T, 2 * G), jnp.bfloat16),
                        pltpu.VMEM((BB * T, 2 * G), jnp.bfloat16)],
        compiler_params=_cparams(("parallel", "arbitrary")),
    )(tok_feats, wcomb, blin, starts, ends)


def _gates(g, cc):
    i_ = jax.nn.sigmoid(g[0])
    f_ = jax.nn.sigmoid(g[1])
    g_ = jnp.tanh(g[2])
    o_ = jax.nn.sigmoid(g[3])
    c2 = f_ * cc + i_ * g_
    h2 = o_ * jnp.tanh(c2)
    return h2, c2


def _lstm0_body(cgf_ref, cgb_ref, h0f_ref, c0f_ref, h0b_ref, c0b_ref,
                bhf_ref, bhb_ref, bf_ref, bb_ref,
                of_ref, ob_ref, hfo_ref, cfo_ref, hbo_ref, cbo_ref,
                hf_s, cf_s, hb_s, cb_s):
    j = pl.program_id(1)

    @pl.when(j == 0)
    def _():
        hf_s[...] = h0f_ref[...]
        cf_s[...] = c0f_ref[...]
        hb_s[...] = h0b_ref[...]
        cb_s[...] = c0b_ref[...]

    def step(tt, carry):
        hf, cf, hb, cb = carry
        rt = TC - 1 - tt
        ghf = jnp.dot(hf, bhf_ref[...], preferred_element_type=_f32) \
            + bf_ref[...]
        h2f, c2f = _gates([cgf_ref[k, tt] + ghf[:, k * SP:k * SP + H]
                           for k in range(4)], cf)
        vf = (j * TC + tt) < C
        hf = jnp.where(vf, h2f, hf)
        cf = jnp.where(vf, c2f, cf)
        of_ref[tt] = hf

        ghb = jnp.dot(hb, bhb_ref[...], preferred_element_type=_f32) \
            + bb_ref[...]
        h2b, c2b = _gates([cgb_ref[k, rt] + ghb[:, k * SP:k * SP + H]
                           for k in range(4)], cb)
        vb = (j * TC + tt) >= (TP - C)
        hb = jnp.where(vb, h2b, hb)
        cb = jnp.where(vb, c2b, cb)
        ob_ref[rt] = hb
        return hf, cf, hb, cb

    hf, cf, hb, cb = jax.lax.fori_loop(
        0, TC, step, (hf_s[...], cf_s[...], hb_s[...], cb_s[...]))
    hf_s[...] = hf
    cf_s[...] = cf
    hb_s[...] = hb
    cb_s[...] = cb
    hfo_ref[...] = hf
    cfo_ref[...] = cf
    hbo_ref[...] = hb
    cbo_ref[...] = cb


def _lstm0(cgf, cgb, h0f, c0f, h0b, c0b, bhf, bhb, biasf, biasb):
    state = jax.ShapeDtypeStruct((B, H), _f32)
    seq = jax.ShapeDtypeStruct((TP, B, H), _f32)
    bspec = pl.BlockSpec((H, GP), lambda i, j: (0, 0))
    sspec = pl.BlockSpec((BK, H), lambda i, j: (i, 0))
    return pl.pallas_call(
        _lstm0_body,
        grid=(2, NCH),
        in_specs=[
            pl.BlockSpec((4, TC, BK, H), lambda i, j: (0, j, i, 0)),
            pl.BlockSpec((4, TC, BK, H), lambda i, j: (0, NCH - 1 - j, i, 0)),
            sspec, sspec, sspec, sspec,
            bspec, bspec,
            pl.BlockSpec((1, GP), lambda i, j: (0, 0)),
            pl.BlockSpec((1, GP), lambda i, j: (0, 0)),
        ],
        out_specs=[
            pl.BlockSpec((TC, BK, H), lambda i, j: (j, i, 0)),
            pl.BlockSpec((TC, BK, H), lambda i, j: (NCH - 1 - j, i, 0)),
            sspec, sspec, sspec, sspec,
        ],
        out_shape=[seq, seq, state, state, state, state],
        scratch_shapes=[pltpu.VMEM((BK, H), _f32) for _ in range(4)],
        compiler_params=_cparams(("parallel", "arbitrary")),
    )(cgf, cgb, h0f, c0f, h0b, c0b, bhf, bhb, biasf, biasb)


def _lstm_body(pf_ref, pb_ref, pfr_ref, pbr_ref,
               h0f_ref, c0f_ref, h0b_ref, c0b_ref,
               a1f_ref, a2f_ref, bhf_ref, a1b_ref, a2b_ref, bhb_ref,
               bf_ref, bb_ref,
               of_ref, ob_ref, hfo_ref, cfo_ref, hbo_ref, cbo_ref,
               hf_s, cf_s, hb_s, cb_s, gxf_s, gxb_s):
    j = pl.program_id(1)

    @pl.when(j == 0)
    def _():
        hf_s[...] = h0f_ref[...]
        cf_s[...] = c0f_ref[...]
        hb_s[...] = h0b_ref[...]
        cb_s[...] = c0b_ref[...]

    zf = pf_ref[...].reshape(TC * BK, H)
    zb = pb_ref[...].reshape(TC * BK, H)
    gxf = (jnp.dot(zf, a1f_ref[...], preferred_element_type=_f32)
           + jnp.dot(zb, a2f_ref[...], preferred_element_type=_f32)
           + bf_ref[...])
    gxf_s[...] = gxf.reshape(TC, BK, GP)
    zfr = pfr_ref[...].reshape(TC * BK, H)
    zbr = pbr_ref[...].reshape(TC * BK, H)
    gxb = (jnp.dot(zfr, a1b_ref[...], preferred_element_type=_f32)
           + jnp.dot(zbr, a2b_ref[...], preferred_element_type=_f32)
           + bb_ref[...])
    gxb_s[...] = gxb.reshape(TC, BK, GP)

    def step(tt, carry):
        hf, cf, hb, cb = carry
        rt = TC - 1 - tt
        ghf = gxf_s[tt] + jnp.dot(hf, bhf_ref[...],
                                  preferred_element_type=_f32)
        h2f, c2f = _gates([ghf[:, k * SP:k * SP + H] for k in range(4)], cf)
        vf = (j * TC + tt) < C
        hf = jnp.where(vf, h2f, hf)
        cf = jnp.where(vf, c2f, cf)
        of_ref[tt] = hf

        ghb = gxb_s[rt] + jnp.dot(hb, bhb_ref[...],
                                  preferred_element_type=_f32)
        h2b, c2b = _gates([ghb[:, k * SP:k * SP + H] for k in range(4)], cb)
        vb = (j * TC + tt) >= (TP - C)
        hb = jnp.where(vb, h2b, hb)
        cb = jnp.where(vb, c2b, cb)
        ob_ref[rt] = hb
        return hf, cf, hb, cb

    hf, cf, hb, cb = jax.lax.fori_loop(
        0, TC, step, (hf_s[...], cf_s[...], hb_s[...], cb_s[...]))
    hf_s[...] = hf
    cf_s[...] = cf
    hb_s[...] = hb
    cb_s[...] = cb
    hfo_ref[...] = hf
    cfo_ref[...] = cf
    hbo_ref[...] = hb
    cbo_ref[...] = cb


def _lstm(pf, pb, h0f, c0f, h0b, c0b, a1f, a2f, bhf, a1b, a2b, bhb,
          biasf, biasb):
    state = jax.ShapeDtypeStruct((B, H), _f32)
    seq = jax.ShapeDtypeStruct((TP, B, H), _f32)
    wspec = pl.BlockSpec((H, GP), lambda i, j: (0, 0))
    sspec = pl.BlockSpec((BK, H), lambda i, j: (i, 0))
    fwd = pl.BlockSpec((TC, BK, H), lambda i, j: (j, i, 0))
    rev = pl.BlockSpec((TC, BK, H), lambda i, j: (NCH - 1 - j, i, 0))
    return pl.pallas_call(
        _lstm_body,
        grid=(2, NCH),
        in_specs=[
            fwd, fwd, rev, rev,
            sspec, sspec, sspec, sspec,
            wspec, wspec, wspec, wspec, wspec, wspec,
            pl.BlockSpec((1, GP), lambda i, j: (0, 0)),
            pl.BlockSpec((1, GP), lambda i, j: (0, 0)),
        ],
        out_specs=[fwd, rev, sspec, sspec, sspec, sspec],
        out_shape=[seq, seq, state, state, state, state],
        scratch_shapes=([pltpu.VMEM((BK, H), _f32) for _ in range(4)]
                        + [pltpu.VMEM((TC, BK, GP), _f32) for _ in range(2)]),
        compiler_params=_cparams(("parallel", "arbitrary")),
    )(pf, pb, pf, pb, h0f, c0f, h0b, c0b, a1f, a2f, bhf, a1b, a2b, bhb,
      biasf, biasb)


def _k4_body(f_ref, b_ref, wf_ref, wb_ref, bias_ref, y_ref):
    y = (jnp.dot(f_ref[...].reshape(TC * BK, H), wf_ref[...],
                 preferred_element_type=_f32)
         + jnp.dot(b_ref[...].reshape(TC * BK, H), wb_ref[...],
                   preferred_element_type=_f32)
         + bias_ref[...])
    y_ref[...] = y.reshape(TC, BK, 2)


def _k4(of, ob, wf, wb, bias):
    fwd = pl.BlockSpec((TC, BK, H), lambda i, j: (j, i, 0))
    return pl.pallas_call(
        _k4_body,
        grid=(2, NCH),
        in_specs=[
            fwd, fwd,
            pl.BlockSpec((H, 2), lambda i, j: (0, 0)),
            pl.BlockSpec((H, 2), lambda i, j: (0, 0)),
            pl.BlockSpec((1, 2), lambda i, j: (0, 0)),
        ],
        out_specs=pl.BlockSpec((TC, BK, 2), lambda i, j: (j, i, 0)),
        out_shape=jax.ShapeDtypeStruct((TP, B, 2), _f32),
        compiler_params=_cparams(("parallel", "arbitrary")),
    )(of, ob, wf, wb, bias)


def _pad_stripes(wt):
    parts = []
    for k in range(4):
        p = wt[:, k * H:(k + 1) * H]
        parts.append(jnp.pad(p, ((0, 0), (0, SP - H))))
    return jnp.concatenate(parts, axis=1)


def kernel(tok_feats, offset_mapping, h0, c0, w_lin, b_lin, w_ih0_f, w_ih0_b,
           w_ih_f, w_ih_b, w_hh_f, w_hh_b, b_f, b_b, w1, b1, w2, b2):
    wlinT = w_lin.T
    wcatT = jnp.concatenate([w_ih0_f, w_ih0_b], axis=0).T
    wcomb = _k1(wlinT, wcatT)
    blin = jnp.concatenate([b_lin @ w_ih0_f.T, b_lin @ w_ih0_b.T])[None, :]
    starts = offset_mapping[..., 0]
    ends = offset_mapping[..., 1]
    cgf, cgb = _k2(tok_feats, wcomb, blin, starts, ends)

    bias_f0 = _pad_stripes((b_f[0])[None, :])
    bias_b0 = _pad_stripes((b_b[0])[None, :])

    h0f = [h0[2 * l] for l in range(L)]
    h0b = [h0[2 * l + 1] for l in range(L)]
    c0f = [c0[2 * l] for l in range(L)]
    c0b = [c0[2 * l + 1] for l in range(L)]

    bhf = [_pad_stripes(w_hh_f[l].T) for l in range(L)]
    bhb = [_pad_stripes(w_hh_b[l].T) for l in range(L)]

    of, ob, hf, cf, hb, cb = _lstm0(cgf, cgb, h0f[0], c0f[0], h0b[0], c0b[0],
                                    bhf[0], bhb[0], bias_f0, bias_b0)
    hs = [hf, hb]
    cs = [cf, cb]
    for l in range(1, L):
        wtf = w_ih_f[l - 1].T
        wtb = w_ih_b[l - 1].T
        a1f = _pad_stripes(wtf[:H])
        a2f = _pad_stripes(wtf[H:])
        a1b = _pad_stripes(wtb[:H])
        a2b = _pad_stripes(wtb[H:])
        biasf = _pad_stripes((b_f[l])[None, :])
        biasb = _pad_stripes((b_b[l])[None, :])
        of, ob, hf, cf, hb, cb = _lstm(of, ob, h0f[l], c0f[l], h0b[l], c0b[l],
                                       a1f, a2f, bhf[l], a1b, a2b, bhb[l],
                                       biasf, biasb)
        hs += [hf, hb]
        cs += [cf, cb]

    w12 = w2 @ w1
    b12 = b2 + b1 @ w2.T
    w12t = w12.T
    y = _k4(of, ob, w12t[:H], w12t[H:], b12[None, :])

    yt = jnp.transpose(y[:C], (1, 0, 2))
    hn = jnp.stack(hs)
    cn = jnp.stack(cs)
    return yt[..., :1], yt[..., 1:], hn, cn


# speedup vs baseline: 1.1606x; 1.1606x over previous
import jax
import jax.numpy as jnp
from jax.experimental import pallas as pl
from jax.experimental.pallas import tpu as pltpu

B, T, C, D = 256, 128, 141, 1536
H = 141
G = 4 * H
L = 4
TP = 144
TC = 16
NCH = TP // TC
SP = 256
GP = 4 * SP
BB = 8
BK = B // 2

_f32 = jnp.float32


def _cparams(sems):
    return pltpu.CompilerParams(dimension_semantics=sems)


def _k1_body(wlinT_ref, wcatT_ref, o_ref):
    o_ref[...] = jnp.dot(wlinT_ref[...], wcatT_ref[...],
                         preferred_element_type=_f32).astype(jnp.bfloat16)


def _k1(wlinT, wcatT):
    return pl.pallas_call(
        _k1_body,
        grid=(1,),
        in_specs=[
            pl.BlockSpec((D, D), lambda i: (0, 0)),
            pl.BlockSpec((D, 2 * G), lambda i: (0, 0)),
        ],
        out_specs=pl.BlockSpec((D, 2 * G), lambda i: (0, 0)),
        out_shape=jax.ShapeDtypeStruct((D, 2 * G), jnp.bfloat16),
        compiler_params=_cparams(("arbitrary",)),
    )(wlinT, wcatT)


def _k2_body(x_ref, wc_ref, blin_ref, st_ref, en_ref, cgf_ref, cgb_ref,
             hi_ref, lo_ref):
    xg = jnp.dot(x_ref[...].reshape(BB * T, D).astype(jnp.bfloat16),
                 wc_ref[...],
                 preferred_element_type=_f32) + blin_ref[...]
    bits = jax.lax.bitcast_convert_type(xg, jnp.int32)
    hi = jax.lax.bitcast_convert_type(
        jnp.bitwise_and(bits, jnp.int32(-65536)), _f32)
    hi_ref[...] = hi.astype(jnp.bfloat16)
    lo_ref[...] = (xg - hi).astype(jnp.bfloat16)
    c_col = jax.lax.broadcasted_iota(jnp.int32, (TP, T), 0)
    for r in range(BB):
        st = st_ref[r, :].reshape(1, T)
        en = en_ref[r, :].reshape(1, T)
        cover = ((st <= c_col) & (c_col < en)).astype(jnp.bfloat16)
        row = (jnp.dot(cover, hi_ref[r * T:(r + 1) * T, :],
                       preferred_element_type=_f32)
               + jnp.dot(cover, lo_ref[r * T:(r + 1) * T, :],
                         preferred_element_type=_f32))
        for k in range(4):
            cgf_ref[k, :, r, :] = row[:, k * H:(k + 1) * H]
            cgb_ref[k, :, r, :] = row[:, G + k * H:G + (k + 1) * H]


def _k2(tok_feats, wcomb, blin, starts, ends):
    nb = B // BB
    return pl.pallas_call(
        _k2_body,
        grid=(2, nb // 2),
        in_specs=[
            pl.BlockSpec((BB, T, D), lambda i, j: (i * (nb // 2) + j, 0, 0)),
            pl.BlockSpec((D, 2 * G), lambda i, j: (0, 0)),
            pl.BlockSpec((1, 2 * G), lambda i, j: (0, 0)),
            pl.BlockSpec((BB, T), lambda i, j: (i * (nb // 2) + j, 0)),
            pl.BlockSpec((BB, T), lambda i, j: (i * (nb // 2) + j, 0)),
        ],
        out_specs=[
            pl.BlockSpec((4, TP, BB, H),
                         lambda i, j: (0, 0, i * (nb // 2) + j, 0)),
            pl.BlockSpec((4, TP, BB, H),
                         lambda i, j: (0, 0, i * (nb // 2) + j, 0)),
        ],
        out_shape=[
            jax.ShapeDtypeStruct((4, TP, B, H), _f32),
            jax.ShapeDtypeStruct((4, TP, B, H), _f32),
        ],
        scratch_shapes=[pltpu.VMEM((BB * T, 2 * G), jnp.bfloat16),
                        pltpu.VMEM((BB * T, 2 * G), jnp.bfloat16)],
        compiler_params=_cparams(("parallel", "arbitrary")),
    )(tok_feats, wcomb, blin, starts, ends)


def _gates(g, cc):
    i_ = jax.nn.sigmoid(g[0])
    f_ = jax.nn.sigmoid(g[1])
    g_ = jnp.tanh(g[2])
    o_ = jax.nn.sigmoid(g[3])
    c2 = f_ * cc + i_ * g_
    h2 = o_ * jnp.tanh(c2)
    return h2, c2


def _lstm0_body(cgf_ref, cgb_ref, h0f_ref, c0f_ref, h0b_ref, c0b_ref,
                bhf_ref, bhb_ref, bf_ref, bb_ref,
                of_ref, ob_ref, hfo_ref, cfo_ref, hbo_ref, cbo_ref,
                hf_s, cf_s, hb_s, cb_s):
    j = pl.program_id(1)

    @pl.when(j == 0)
    def _():
        hf_s[...] = h0f_ref[...]
        cf_s[...] = c0f_ref[...]
        hb_s[...] = h0b_ref[...]
        cb_s[...] = c0b_ref[...]

    def step(tt, carry):
        hf, cf, hb, cb = carry
        rt = TC - 1 - tt
        ghf = jnp.dot(hf, bhf_ref[...], preferred_element_type=_f32) \
            + bf_ref[...]
        h2f, c2f = _gates([cgf_ref[k, tt] + ghf[:, k * SP:k * SP + H]
                           for k in range(4)], cf)
        vf = (j * TC + tt) < C
        hf = jnp.where(vf, h2f, hf)
        cf = jnp.where(vf, c2f, cf)
        of_ref[tt] = hf

        ghb = jnp.dot(hb, bhb_ref[...], preferred_element_type=_f32) \
            + bb_ref[...]
        h2b, c2b = _gates([cgb_ref[k, rt] + ghb[:, k * SP:k * SP + H]
                           for k in range(4)], cb)
        vb = (j * TC + tt) >= (TP - C)
        hb = jnp.where(vb, h2b, hb)
        cb = jnp.where(vb, c2b, cb)
        ob_ref[rt] = hb
        return hf, cf, hb, cb

    carry = (hf_s[...], cf_s[...], hb_s[...], cb_s[...])
    for tt in range(TC):
        carry = step(tt, carry)
    hf, cf, hb, cb = carry
    hf_s[...] = hf
    cf_s[...] = cf
    hb_s[...] = hb
    cb_s[...] = cb
    hfo_ref[...] = hf
    cfo_ref[...] = cf
    hbo_ref[...] = hb
    cbo_ref[...] = cb


def _lstm0(cgf, cgb, h0f, c0f, h0b, c0b, bhf, bhb, biasf, biasb):
    state = jax.ShapeDtypeStruct((B, H), _f32)
    seq = jax.ShapeDtypeStruct((TP, B, H), _f32)
    bspec = pl.BlockSpec((H, GP), lambda i, j: (0, 0))
    sspec = pl.BlockSpec((BK, H), lambda i, j: (i, 0))
    return pl.pallas_call(
        _lstm0_body,
        grid=(2, NCH),
        in_specs=[
            pl.BlockSpec((4, TC, BK, H), lambda i, j: (0, j, i, 0)),
            pl.BlockSpec((4, TC, BK, H), lambda i, j: (0, NCH - 1 - j, i, 0)),
            sspec, sspec, sspec, sspec,
            bspec, bspec,
            pl.BlockSpec((1, GP), lambda i, j: (0, 0)),
            pl.BlockSpec((1, GP), lambda i, j: (0, 0)),
        ],
        out_specs=[
            pl.BlockSpec((TC, BK, H), lambda i, j: (j, i, 0)),
            pl.BlockSpec((TC, BK, H), lambda i, j: (NCH - 1 - j, i, 0)),
            sspec, sspec, sspec, sspec,
        ],
        out_shape=[seq, seq, state, state, state, state],
        scratch_shapes=[pltpu.VMEM((BK, H), _f32) for _ in range(4)],
        compiler_params=_cparams(("parallel", "arbitrary")),
    )(cgf, cgb, h0f, c0f, h0b, c0b, bhf, bhb, biasf, biasb)


def _lstm_body(pf_ref, pb_ref, pfr_ref, pbr_ref,
               h0f_ref, c0f_ref, h0b_ref, c0b_ref,
               a1f_ref, a2f_ref, bhf_ref, a1b_ref, a2b_ref, bhb_ref,
               bf_ref, bb_ref,
               of_ref, ob_ref, hfo_ref, cfo_ref, hbo_ref, cbo_ref,
               hf_s, cf_s, hb_s, cb_s, gxf_s, gxb_s):
    j = pl.program_id(1)

    @pl.when(j == 0)
    def _():
        hf_s[...] = h0f_ref[...]
        cf_s[...] = c0f_ref[...]
        hb_s[...] = h0b_ref[...]
        cb_s[...] = c0b_ref[...]

    zf = pf_ref[...].reshape(TC * BK, H)
    zb = pb_ref[...].reshape(TC * BK, H)
    gxf = (jnp.dot(zf, a1f_ref[...], preferred_element_type=_f32)
           + jnp.dot(zb, a2f_ref[...], preferred_element_type=_f32)
           + bf_ref[...])
    gxf_s[...] = gxf.reshape(TC, BK, GP)
    zfr = pfr_ref[...].reshape(TC * BK, H)
    zbr = pbr_ref[...].reshape(TC * BK, H)
    gxb = (jnp.dot(zfr, a1b_ref[...], preferred_element_type=_f32)
           + jnp.dot(zbr, a2b_ref[...], preferred_element_type=_f32)
           + bb_ref[...])
    gxb_s[...] = gxb.reshape(TC, BK, GP)

    def step(tt, carry):
        hf, cf, hb, cb = carry
        rt = TC - 1 - tt
        ghf = gxf_s[tt] + jnp.dot(hf, bhf_ref[...],
                                  preferred_element_type=_f32)
        h2f, c2f = _gates([ghf[:, k * SP:k * SP + H] for k in range(4)], cf)
        vf = (j * TC + tt) < C
        hf = jnp.where(vf, h2f, hf)
        cf = jnp.where(vf, c2f, cf)
        of_ref[tt] = hf

        ghb = gxb_s[rt] + jnp.dot(hb, bhb_ref[...],
                                  preferred_element_type=_f32)
        h2b, c2b = _gates([ghb[:, k * SP:k * SP + H] for k in range(4)], cb)
        vb = (j * TC + tt) >= (TP - C)
        hb = jnp.where(vb, h2b, hb)
        cb = jnp.where(vb, c2b, cb)
        ob_ref[rt] = hb
        return hf, cf, hb, cb

    carry = (hf_s[...], cf_s[...], hb_s[...], cb_s[...])
    for tt in range(TC):
        carry = step(tt, carry)
    hf, cf, hb, cb = carry
    hf_s[...] = hf
    cf_s[...] = cf
    hb_s[...] = hb
    cb_s[...] = cb
    hfo_ref[...] = hf
    cfo_ref[...] = cf
    hbo_ref[...] = hb
    cbo_ref[...] = cb


def _lstm(pf, pb, h0f, c0f, h0b, c0b, a1f, a2f, bhf, a1b, a2b, bhb,
          biasf, biasb):
    state = jax.ShapeDtypeStruct((B, H), _f32)
    seq = jax.ShapeDtypeStruct((TP, B, H), _f32)
    wspec = pl.BlockSpec((H, GP), lambda i, j: (0, 0))
    sspec = pl.BlockSpec((BK, H), lambda i, j: (i, 0))
    fwd = pl.BlockSpec((TC, BK, H), lambda i, j: (j, i, 0))
    rev = pl.BlockSpec((TC, BK, H), lambda i, j: (NCH - 1 - j, i, 0))
    return pl.pallas_call(
        _lstm_body,
        grid=(2, NCH),
        in_specs=[
            fwd, fwd, rev, rev,
            sspec, sspec, sspec, sspec,
            wspec, wspec, wspec, wspec, wspec, wspec,
            pl.BlockSpec((1, GP), lambda i, j: (0, 0)),
            pl.BlockSpec((1, GP), lambda i, j: (0, 0)),
        ],
        out_specs=[fwd, rev, sspec, sspec, sspec, sspec],
        out_shape=[seq, seq, state, state, state, state],
        scratch_shapes=([pltpu.VMEM((BK, H), _f32) for _ in range(4)]
                        + [pltpu.VMEM((TC, BK, GP), _f32) for _ in range(2)]),
        compiler_params=_cparams(("parallel", "arbitrary")),
    )(pf, pb, pf, pb, h0f, c0f, h0b, c0b, a1f, a2f, bhf, a1b, a2b, bhb,
      biasf, biasb)


def _k4_body(f_ref, b_ref, wf_ref, wb_ref, bias_ref, y_ref):
    y = (jnp.dot(f_ref[...].reshape(TC * BK, H), wf_ref[...],
                 preferred_element_type=_f32)
         + jnp.dot(b_ref[...].reshape(TC * BK, H), wb_ref[...],
                   preferred_element_type=_f32)
         + bias_ref[...])
    y_ref[...] = y.reshape(TC, BK, 2)


def _k4(of, ob, wf, wb, bias):
    fwd = pl.BlockSpec((TC, BK, H), lambda i, j: (j, i, 0))
    return pl.pallas_call(
        _k4_body,
        grid=(2, NCH),
        in_specs=[
            fwd, fwd,
            pl.BlockSpec((H, 2), lambda i, j: (0, 0)),
            pl.BlockSpec((H, 2), lambda i, j: (0, 0)),
            pl.BlockSpec((1, 2), lambda i, j: (0, 0)),
        ],
        out_specs=pl.BlockSpec((TC, BK, 2), lambda i, j: (j, i, 0)),
        out_shape=jax.ShapeDtypeStruct((TP, B, 2), _f32),
        compiler_params=_cparams(("parallel", "arbitrary")),
    )(of, ob, wf, wb, bias)


def _pad_stripes(wt):
    parts = []
    for k in range(4):
        p = wt[:, k * H:(k + 1) * H]
        parts.append(jnp.pad(p, ((0, 0), (0, SP - H))))
    return jnp.concatenate(parts, axis=1)


def kernel(tok_feats, offset_mapping, h0, c0, w_lin, b_lin, w_ih0_f, w_ih0_b,
           w_ih_f, w_ih_b, w_hh_f, w_hh_b, b_f, b_b, w1, b1, w2, b2):
    wlinT = w_lin.T
    wcatT = jnp.concatenate([w_ih0_f, w_ih0_b], axis=0).T
    wcomb = _k1(wlinT, wcatT)
    blin = jnp.concatenate([b_lin @ w_ih0_f.T, b_lin @ w_ih0_b.T])[None, :]
    starts = offset_mapping[..., 0]
    ends = offset_mapping[..., 1]
    cgf, cgb = _k2(tok_feats, wcomb, blin, starts, ends)

    bias_f0 = _pad_stripes((b_f[0])[None, :])
    bias_b0 = _pad_stripes((b_b[0])[None, :])

    h0f = [h0[2 * l] for l in range(L)]
    h0b = [h0[2 * l + 1] for l in range(L)]
    c0f = [c0[2 * l] for l in range(L)]
    c0b = [c0[2 * l + 1] for l in range(L)]

    bhf = [_pad_stripes(w_hh_f[l].T) for l in range(L)]
    bhb = [_pad_stripes(w_hh_b[l].T) for l in range(L)]

    of, ob, hf, cf, hb, cb = _lstm0(cgf, cgb, h0f[0], c0f[0], h0b[0], c0b[0],
                                    bhf[0], bhb[0], bias_f0, bias_b0)
    hs = [hf, hb]
    cs = [cf, cb]
    for l in range(1, L):
        wtf = w_ih_f[l - 1].T
        wtb = w_ih_b[l - 1].T
        a1f = _pad_stripes(wtf[:H])
        a2f = _pad_stripes(wtf[H:])
        a1b = _pad_stripes(wtb[:H])
        a2b = _pad_stripes(wtb[H:])
        biasf = _pad_stripes((b_f[l])[None, :])
        biasb = _pad_stripes((b_b[l])[None, :])
        of, ob, hf, cf, hb, cb = _lstm(of, ob, h0f[l], c0f[l], h0b[l], c0b[l],
                                       a1f, a2f, bhf[l], a1b, a2b, bhb[l],
                                       biasf, biasb)
        hs += [hf, hb]
        cs += [cf, cb]

    w12 = w2 @ w1
    b12 = b2 + b1 @ w2.T
    w12t = w12.T
    y = _k4(of, ob, w12t[:H], w12t[H:], b12[None, :])

    yt = jnp.transpose(y[:C], (1, 0, 2))
    hn = jnp.stack(hs)
    cn = jnp.stack(cs)
    return yt[..., :1], yt[..., 1:], hn, cn


# speedup vs baseline: 1.1831x; 1.0193x over previous
import jax
import jax.numpy as jnp
from jax.experimental import pallas as pl
from jax.experimental.pallas import tpu as pltpu

B, T, C, D = 256, 128, 141, 1536
H = 141
G = 4 * H
L = 4
TP = 144
TC = 16
NCH = TP // TC
SP = 256
GP = 4 * SP
BB = 8
BK = B // 2

_f32 = jnp.float32


def _cparams(sems):
    return pltpu.CompilerParams(dimension_semantics=sems)


def _k1_body(wlinT_ref, wcatT_ref, o_ref):
    o_ref[...] = jnp.dot(wlinT_ref[...], wcatT_ref[...],
                         preferred_element_type=_f32).astype(jnp.bfloat16)


def _k1(wlinT, wcatT):
    return pl.pallas_call(
        _k1_body,
        grid=(1,),
        in_specs=[
            pl.BlockSpec((D, D), lambda i: (0, 0)),
            pl.BlockSpec((D, 2 * G), lambda i: (0, 0)),
        ],
        out_specs=pl.BlockSpec((D, 2 * G), lambda i: (0, 0)),
        out_shape=jax.ShapeDtypeStruct((D, 2 * G), jnp.bfloat16),
        compiler_params=_cparams(("arbitrary",)),
    )(wlinT, wcatT)


def _k2_body(x_ref, wc_ref, blin_ref, st_ref, en_ref, cgf_ref, cgb_ref,
             hi_ref, lo_ref):
    xg = jnp.dot(x_ref[...].reshape(BB * T, D).astype(jnp.bfloat16),
                 wc_ref[...],
                 preferred_element_type=_f32) + blin_ref[...]
    bits = jax.lax.bitcast_convert_type(xg, jnp.int32)
    hi = jax.lax.bitcast_convert_type(
        jnp.bitwise_and(bits, jnp.int32(-65536)), _f32)
    hi_ref[...] = hi.astype(jnp.bfloat16)
    lo_ref[...] = (xg - hi).astype(jnp.bfloat16)
    c_col = jax.lax.broadcasted_iota(jnp.int32, (TP, T), 0)
    for r in range(BB):
        st = st_ref[r, :].reshape(1, T)
        en = en_ref[r, :].reshape(1, T)
        cover = ((st <= c_col) & (c_col < en)).astype(jnp.bfloat16)
        row = (jnp.dot(cover, hi_ref[r * T:(r + 1) * T, :],
                       preferred_element_type=_f32)
               + jnp.dot(cover, lo_ref[r * T:(r + 1) * T, :],
                         preferred_element_type=_f32))
        for k in range(4):
            cgf_ref[k, :, r, :] = row[:, k * H:(k + 1) * H]
            cgb_ref[k, :, r, :] = row[:, G + k * H:G + (k + 1) * H]


def _k2(tok_feats, wcomb, blin, starts, ends):
    nb = B // BB
    return pl.pallas_call(
        _k2_body,
        grid=(2, nb // 2),
        in_specs=[
            pl.BlockSpec((BB, T, D), lambda i, j: (i * (nb // 2) + j, 0, 0)),
            pl.BlockSpec((D, 2 * G), lambda i, j: (0, 0)),
            pl.BlockSpec((1, 2 * G), lambda i, j: (0, 0)),
            pl.BlockSpec((BB, T), lambda i, j: (i * (nb // 2) + j, 0)),
            pl.BlockSpec((BB, T), lambda i, j: (i * (nb // 2) + j, 0)),
        ],
        out_specs=[
            pl.BlockSpec((4, TP, BB, H),
                         lambda i, j: (0, 0, i * (nb // 2) + j, 0)),
            pl.BlockSpec((4, TP, BB, H),
                         lambda i, j: (0, 0, i * (nb // 2) + j, 0)),
        ],
        out_shape=[
            jax.ShapeDtypeStruct((4, TP, B, H), _f32),
            jax.ShapeDtypeStruct((4, TP, B, H), _f32),
        ],
        scratch_shapes=[pltpu.VMEM((BB * T, 2 * G), jnp.bfloat16),
                        pltpu.VMEM((BB * T, 2 * G), jnp.bfloat16)],
        compiler_params=_cparams(("parallel", "arbitrary")),
    )(tok_feats, wcomb, blin, starts, ends)


def _sig(x):
    return 0.5 * jnp.tanh(0.5 * x) + 0.5


def _gates(g, cc):
    i_ = _sig(g[0])
    f_ = _sig(g[1])
    g_ = jnp.tanh(g[2])
    o_ = _sig(g[3])
    c2 = f_ * cc + i_ * g_
    h2 = o_ * jnp.tanh(c2)
    return h2, c2


def _lstm0_body(cgf_ref, cgb_ref, h0f_ref, c0f_ref, h0b_ref, c0b_ref,
                bhf_ref, bhb_ref, bf_ref, bb_ref,
                of_ref, ob_ref, hfo_ref, cfo_ref, hbo_ref, cbo_ref,
                hf_s, cf_s, hb_s, cb_s):
    j = pl.program_id(1)

    @pl.when(j == 0)
    def _():
        hf_s[...] = h0f_ref[...]
        cf_s[...] = c0f_ref[...]
        hb_s[...] = h0b_ref[...]
        cb_s[...] = c0b_ref[...]

    def step(tt, carry):
        hf, cf, hb, cb = carry
        rt = TC - 1 - tt
        ghf = jnp.dot(hf, bhf_ref[...], preferred_element_type=_f32) \
            + bf_ref[...]
        h2f, c2f = _gates([cgf_ref[k, tt] + ghf[:, k * SP:k * SP + H]
                           for k in range(4)], cf)
        vf = (j * TC + tt) < C
        hf = jnp.where(vf, h2f, hf)
        cf = jnp.where(vf, c2f, cf)
        of_ref[tt] = hf

        ghb = jnp.dot(hb, bhb_ref[...], preferred_element_type=_f32) \
            + bb_ref[...]
        h2b, c2b = _gates([cgb_ref[k, rt] + ghb[:, k * SP:k * SP + H]
                           for k in range(4)], cb)
        vb = (j * TC + tt) >= (TP - C)
        hb = jnp.where(vb, h2b, hb)
        cb = jnp.where(vb, c2b, cb)
        ob_ref[rt] = hb
        return hf, cf, hb, cb

    carry = (hf_s[...], cf_s[...], hb_s[...], cb_s[...])
    for tt in range(TC):
        carry = step(tt, carry)
    hf, cf, hb, cb = carry
    hf_s[...] = hf
    cf_s[...] = cf
    hb_s[...] = hb
    cb_s[...] = cb
    hfo_ref[...] = hf
    cfo_ref[...] = cf
    hbo_ref[...] = hb
    cbo_ref[...] = cb


def _lstm0(cgf, cgb, h0f, c0f, h0b, c0b, bhf, bhb, biasf, biasb):
    state = jax.ShapeDtypeStruct((B, H), _f32)
    seq = jax.ShapeDtypeStruct((TP, B, H), _f32)
    bspec = pl.BlockSpec((H, GP), lambda i, j: (0, 0))
    sspec = pl.BlockSpec((BK, H), lambda i, j: (i, 0))
    return pl.pallas_call(
        _lstm0_body,
        grid=(2, NCH),
        in_specs=[
            pl.BlockSpec((4, TC, BK, H), lambda i, j: (0, j, i, 0)),
            pl.BlockSpec((4, TC, BK, H), lambda i, j: (0, NCH - 1 - j, i, 0)),
            sspec, sspec, sspec, sspec,
            bspec, bspec,
            pl.BlockSpec((1, GP), lambda i, j: (0, 0)),
            pl.BlockSpec((1, GP), lambda i, j: (0, 0)),
        ],
        out_specs=[
            pl.BlockSpec((TC, BK, H), lambda i, j: (j, i, 0)),
            pl.BlockSpec((TC, BK, H), lambda i, j: (NCH - 1 - j, i, 0)),
            sspec, sspec, sspec, sspec,
        ],
        out_shape=[seq, seq, state, state, state, state],
        scratch_shapes=[pltpu.VMEM((BK, H), _f32) for _ in range(4)],
        compiler_params=_cparams(("parallel", "arbitrary")),
    )(cgf, cgb, h0f, c0f, h0b, c0b, bhf, bhb, biasf, biasb)


def _lstm_body(pf_ref, pb_ref, pfr_ref, pbr_ref,
               h0f_ref, c0f_ref, h0b_ref, c0b_ref,
               a1f_ref, a2f_ref, bhf_ref, a1b_ref, a2b_ref, bhb_ref,
               bf_ref, bb_ref,
               of_ref, ob_ref, hfo_ref, cfo_ref, hbo_ref, cbo_ref,
               hf_s, cf_s, hb_s, cb_s, gxf_s, gxb_s):
    j = pl.program_id(1)

    @pl.when(j == 0)
    def _():
        hf_s[...] = h0f_ref[...]
        cf_s[...] = c0f_ref[...]
        hb_s[...] = h0b_ref[...]
        cb_s[...] = c0b_ref[...]

    zf = pf_ref[...].reshape(TC * BK, H)
    zb = pb_ref[...].reshape(TC * BK, H)
    gxf = (jnp.dot(zf, a1f_ref[...], preferred_element_type=_f32)
           + jnp.dot(zb, a2f_ref[...], preferred_element_type=_f32)
           + bf_ref[...])
    gxf_s[...] = gxf.reshape(TC, BK, GP)
    zfr = pfr_ref[...].reshape(TC * BK, H)
    zbr = pbr_ref[...].reshape(TC * BK, H)
    gxb = (jnp.dot(zfr, a1b_ref[...], preferred_element_type=_f32)
           + jnp.dot(zbr, a2b_ref[...], preferred_element_type=_f32)
           + bb_ref[...])
    gxb_s[...] = gxb.reshape(TC, BK, GP)

    def step(tt, carry):
        hf, cf, hb, cb = carry
        rt = TC - 1 - tt
        ghf = gxf_s[tt] + jnp.dot(hf, bhf_ref[...],
                                  preferred_element_type=_f32)
        h2f, c2f = _gates([ghf[:, k * SP:k * SP + H] for k in range(4)], cf)
        vf = (j * TC + tt) < C
        hf = jnp.where(vf, h2f, hf)
        cf = jnp.where(vf, c2f, cf)
        of_ref[tt] = hf

        ghb = gxb_s[rt] + jnp.dot(hb, bhb_ref[...],
                                  preferred_element_type=_f32)
        h2b, c2b = _gates([ghb[:, k * SP:k * SP + H] for k in range(4)], cb)
        vb = (j * TC + tt) >= (TP - C)
        hb = jnp.where(vb, h2b, hb)
        cb = jnp.where(vb, c2b, cb)
        ob_ref[rt] = hb
        return hf, cf, hb, cb

    carry = (hf_s[...], cf_s[...], hb_s[...], cb_s[...])
    for tt in range(TC):
        carry = step(tt, carry)
    hf, cf, hb, cb = carry
    hf_s[...] = hf
    cf_s[...] = cf
    hb_s[...] = hb
    cb_s[...] = cb
    hfo_ref[...] = hf
    cfo_ref[...] = cf
    hbo_ref[...] = hb
    cbo_ref[...] = cb


def _lstm(pf, pb, h0f, c0f, h0b, c0b, a1f, a2f, bhf, a1b, a2b, bhb,
          biasf, biasb):
    state = jax.ShapeDtypeStruct((B, H), _f32)
    seq = jax.ShapeDtypeStruct((TP, B, H), _f32)
    wspec = pl.BlockSpec((H, GP), lambda i, j: (0, 0))
    sspec = pl.BlockSpec((BK, H), lambda i, j: (i, 0))
    fwd = pl.BlockSpec((TC, BK, H), lambda i, j: (j, i, 0))
    rev = pl.BlockSpec((TC, BK, H), lambda i, j: (NCH - 1 - j, i, 0))
    return pl.pallas_call(
        _lstm_body,
        grid=(2, NCH),
        in_specs=[
            fwd, fwd, rev, rev,
            sspec, sspec, sspec, sspec,
            wspec, wspec, wspec, wspec, wspec, wspec,
            pl.BlockSpec((1, GP), lambda i, j: (0, 0)),
            pl.BlockSpec((1, GP), lambda i, j: (0, 0)),
        ],
        out_specs=[fwd, rev, sspec, sspec, sspec, sspec],
        out_shape=[seq, seq, state, state, state, state],
        scratch_shapes=([pltpu.VMEM((BK, H), _f32) for _ in range(4)]
                        + [pltpu.VMEM((TC, BK, GP), _f32) for _ in range(2)]),
        compiler_params=_cparams(("parallel", "arbitrary")),
    )(pf, pb, pf, pb, h0f, c0f, h0b, c0b, a1f, a2f, bhf, a1b, a2b, bhb,
      biasf, biasb)


def _k4_body(f_ref, b_ref, wf_ref, wb_ref, bias_ref, y_ref):
    y = (jnp.dot(f_ref[...].reshape(TC * BK, H), wf_ref[...],
                 preferred_element_type=_f32)
         + jnp.dot(b_ref[...].reshape(TC * BK, H), wb_ref[...],
                   preferred_element_type=_f32)
         + bias_ref[...])
    y_ref[...] = y.reshape(TC, BK, 2)


def _k4(of, ob, wf, wb, bias):
    fwd = pl.BlockSpec((TC, BK, H), lambda i, j: (j, i, 0))
    return pl.pallas_call(
        _k4_body,
        grid=(2, NCH),
        in_specs=[
            fwd, fwd,
            pl.BlockSpec((H, 2), lambda i, j: (0, 0)),
            pl.BlockSpec((H, 2), lambda i, j: (0, 0)),
            pl.BlockSpec((1, 2), lambda i, j: (0, 0)),
        ],
        out_specs=pl.BlockSpec((TC, BK, 2), lambda i, j: (j, i, 0)),
        out_shape=jax.ShapeDtypeStruct((TP, B, 2), _f32),
        compiler_params=_cparams(("parallel", "arbitrary")),
    )(of, ob, wf, wb, bias)


def _pad_stripes(wt):
    parts = []
    for k in range(4):
        p = wt[:, k * H:(k + 1) * H]
        parts.append(jnp.pad(p, ((0, 0), (0, SP - H))))
    return jnp.concatenate(parts, axis=1)


def kernel(tok_feats, offset_mapping, h0, c0, w_lin, b_lin, w_ih0_f, w_ih0_b,
           w_ih_f, w_ih_b, w_hh_f, w_hh_b, b_f, b_b, w1, b1, w2, b2):
    wlinT = w_lin.T
    wcatT = jnp.concatenate([w_ih0_f, w_ih0_b], axis=0).T
    wcomb = _k1(wlinT, wcatT)
    blin = jnp.concatenate([b_lin @ w_ih0_f.T, b_lin @ w_ih0_b.T])[None, :]
    starts = offset_mapping[..., 0]
    ends = offset_mapping[..., 1]
    cgf, cgb = _k2(tok_feats, wcomb, blin, starts, ends)

    bias_f0 = _pad_stripes((b_f[0])[None, :])
    bias_b0 = _pad_stripes((b_b[0])[None, :])

    h0f = [h0[2 * l] for l in range(L)]
    h0b = [h0[2 * l + 1] for l in range(L)]
    c0f = [c0[2 * l] for l in range(L)]
    c0b = [c0[2 * l + 1] for l in range(L)]

    bhf = [_pad_stripes(w_hh_f[l].T) for l in range(L)]
    bhb = [_pad_stripes(w_hh_b[l].T) for l in range(L)]

    of, ob, hf, cf, hb, cb = _lstm0(cgf, cgb, h0f[0], c0f[0], h0b[0], c0b[0],
                                    bhf[0], bhb[0], bias_f0, bias_b0)
    hs = [hf, hb]
    cs = [cf, cb]
    for l in range(1, L):
        wtf = w_ih_f[l - 1].T
        wtb = w_ih_b[l - 1].T
        a1f = _pad_stripes(wtf[:H])
        a2f = _pad_stripes(wtf[H:])
        a1b = _pad_stripes(wtb[:H])
        a2b = _pad_stripes(wtb[H:])
        biasf = _pad_stripes((b_f[l])[None, :])
        biasb = _pad_stripes((b_b[l])[None, :])
        of, ob, hf, cf, hb, cb = _lstm(of, ob, h0f[l], c0f[l], h0b[l], c0b[l],
                                       a1f, a2f, bhf[l], a1b, a2b, bhb[l],
                                       biasf, biasb)
        hs += [hf, hb]
        cs += [cf, cb]

    w12 = w2 @ w1
    b12 = b2 + b1 @ w2.T
    w12t = w12.T
    y = _k4(of, ob, w12t[:H], w12t[H:], b12[None, :])

    yt = jnp.transpose(y[:C], (1, 0, 2))
    hn = jnp.stack(hs)
    cn = jnp.stack(cs)
    return yt[..., :1], yt[..., 1:], hn, cn


# speedup vs baseline: 1.2041x; 1.0177x over previous
import jax
import jax.numpy as jnp
from jax.experimental import pallas as pl
from jax.experimental.pallas import tpu as pltpu

B, T, C, D = 256, 128, 141, 1536
H = 141
G = 4 * H
L = 4
TP = 144
TC = 16
NCH = TP // TC
SP = 256
GP = 4 * SP
BB = 8
BK = B // 2

_f32 = jnp.float32


def _cparams(sems):
    return pltpu.CompilerParams(dimension_semantics=sems)


def _k1_body(wlinT_ref, wcatT_ref, o_ref):
    o_ref[...] = jnp.dot(wlinT_ref[...], wcatT_ref[...],
                         preferred_element_type=_f32).astype(jnp.bfloat16)


def _k1(wlinT, wcatT):
    return pl.pallas_call(
        _k1_body,
        grid=(1,),
        in_specs=[
            pl.BlockSpec((D, D), lambda i: (0, 0)),
            pl.BlockSpec((D, 2 * G), lambda i: (0, 0)),
        ],
        out_specs=pl.BlockSpec((D, 2 * G), lambda i: (0, 0)),
        out_shape=jax.ShapeDtypeStruct((D, 2 * G), jnp.bfloat16),
        compiler_params=_cparams(("arbitrary",)),
    )(wlinT, wcatT)


def _k2_body(x_ref, wc_ref, blin_ref, st_ref, en_ref, cgf_ref, cgb_ref,
             hi_ref, lo_ref):
    xg = jnp.dot(x_ref[...].reshape(BB * T, D).astype(jnp.bfloat16),
                 wc_ref[...],
                 preferred_element_type=_f32) + blin_ref[...]
    bits = jax.lax.bitcast_convert_type(xg, jnp.int32)
    hi = jax.lax.bitcast_convert_type(
        jnp.bitwise_and(bits, jnp.int32(-65536)), _f32)
    hi_ref[...] = hi.astype(jnp.bfloat16)
    lo_ref[...] = (xg - hi).astype(jnp.bfloat16)
    c_col = jax.lax.broadcasted_iota(jnp.int32, (TP, T), 0)
    for r in range(BB):
        st = st_ref[r, :].reshape(1, T)
        en = en_ref[r, :].reshape(1, T)
        cover = ((st <= c_col) & (c_col < en)).astype(jnp.bfloat16)
        row = (jnp.dot(cover, hi_ref[r * T:(r + 1) * T, :],
                       preferred_element_type=_f32)
               + jnp.dot(cover, lo_ref[r * T:(r + 1) * T, :],
                         preferred_element_type=_f32))
        for k in range(4):
            cgf_ref[k, :, r, :] = row[:, k * H:(k + 1) * H]
            cgb_ref[k, :, r, :] = row[:, G + k * H:G + (k + 1) * H]


def _k2(tok_feats, wcomb, blin, starts, ends):
    nb = B // BB
    return pl.pallas_call(
        _k2_body,
        grid=(2, nb // 2),
        in_specs=[
            pl.BlockSpec((BB, T, D), lambda i, j: (i * (nb // 2) + j, 0, 0)),
            pl.BlockSpec((D, 2 * G), lambda i, j: (0, 0)),
            pl.BlockSpec((1, 2 * G), lambda i, j: (0, 0)),
            pl.BlockSpec((BB, T), lambda i, j: (i * (nb // 2) + j, 0)),
            pl.BlockSpec((BB, T), lambda i, j: (i * (nb // 2) + j, 0)),
        ],
        out_specs=[
            pl.BlockSpec((4, TP, BB, H),
                         lambda i, j: (0, 0, i * (nb // 2) + j, 0)),
            pl.BlockSpec((4, TP, BB, H),
                         lambda i, j: (0, 0, i * (nb // 2) + j, 0)),
        ],
        out_shape=[
            jax.ShapeDtypeStruct((4, TP, B, H), _f32),
            jax.ShapeDtypeStruct((4, TP, B, H), _f32),
        ],
        scratch_shapes=[pltpu.VMEM((BB * T, 2 * G), jnp.bfloat16),
                        pltpu.VMEM((BB * T, 2 * G), jnp.bfloat16)],
        compiler_params=_cparams(("parallel", "arbitrary")),
    )(tok_feats, wcomb, blin, starts, ends)


def _sig(x):
    return 0.5 * jnp.tanh(0.5 * x) + 0.5


def _gates(g, cc):
    i_ = _sig(g[0])
    f_ = _sig(g[1])
    g_ = jnp.tanh(g[2])
    o_ = _sig(g[3])
    c2 = f_ * cc + i_ * g_
    h2 = o_ * jnp.tanh(c2)
    return h2, c2


def _lstm0_body(cgf_ref, cgb_ref, h0f_ref, c0f_ref, h0b_ref, c0b_ref,
                bhf_ref, bhb_ref, bf_ref, bb_ref,
                of_ref, ob_ref, hfo_ref, cfo_ref, hbo_ref, cbo_ref,
                hf_s, cf_s, hb_s, cb_s):
    j = pl.program_id(1)

    @pl.when(j == 0)
    def _():
        hf_s[...] = h0f_ref[...]
        cf_s[...] = c0f_ref[...]
        hb_s[...] = h0b_ref[...]
        cb_s[...] = c0b_ref[...]

    def step(tt, carry):
        hf, cf, hb, cb = carry
        rt = TC - 1 - tt
        ghf = jnp.dot(hf, bhf_ref[...], preferred_element_type=_f32) \
            + bf_ref[...]
        h2f, c2f = _gates([cgf_ref[k, tt] + ghf[:, k * SP:k * SP + H]
                           for k in range(4)], cf)
        vf = (j * TC + tt) < C
        hf = jnp.where(vf, h2f, hf)
        cf = jnp.where(vf, c2f, cf)
        of_ref[tt] = hf.astype(jnp.bfloat16)

        ghb = jnp.dot(hb, bhb_ref[...], preferred_element_type=_f32) \
            + bb_ref[...]
        h2b, c2b = _gates([cgb_ref[k, rt] + ghb[:, k * SP:k * SP + H]
                           for k in range(4)], cb)
        vb = (j * TC + tt) >= (TP - C)
        hb = jnp.where(vb, h2b, hb)
        cb = jnp.where(vb, c2b, cb)
        ob_ref[rt] = hb.astype(jnp.bfloat16)
        return hf, cf, hb, cb

    carry = (hf_s[...], cf_s[...], hb_s[...], cb_s[...])
    for tt in range(TC):
        carry = step(tt, carry)
    hf, cf, hb, cb = carry
    hf_s[...] = hf
    cf_s[...] = cf
    hb_s[...] = hb
    cb_s[...] = cb
    hfo_ref[...] = hf
    cfo_ref[...] = cf
    hbo_ref[...] = hb
    cbo_ref[...] = cb


def _lstm0(cgf, cgb, h0f, c0f, h0b, c0b, bhf, bhb, biasf, biasb):
    state = jax.ShapeDtypeStruct((B, H), _f32)
    seq = jax.ShapeDtypeStruct((TP, B, H), jnp.bfloat16)
    bspec = pl.BlockSpec((H, GP), lambda i, j: (0, 0))
    sspec = pl.BlockSpec((BK, H), lambda i, j: (i, 0))
    return pl.pallas_call(
        _lstm0_body,
        grid=(2, NCH),
        in_specs=[
            pl.BlockSpec((4, TC, BK, H), lambda i, j: (0, j, i, 0)),
            pl.BlockSpec((4, TC, BK, H), lambda i, j: (0, NCH - 1 - j, i, 0)),
            sspec, sspec, sspec, sspec,
            bspec, bspec,
            pl.BlockSpec((1, GP), lambda i, j: (0, 0)),
            pl.BlockSpec((1, GP), lambda i, j: (0, 0)),
        ],
        out_specs=[
            pl.BlockSpec((TC, BK, H), lambda i, j: (j, i, 0)),
            pl.BlockSpec((TC, BK, H), lambda i, j: (NCH - 1 - j, i, 0)),
            sspec, sspec, sspec, sspec,
        ],
        out_shape=[seq, seq, state, state, state, state],
        scratch_shapes=[pltpu.VMEM((BK, H), _f32) for _ in range(4)],
        compiler_params=_cparams(("parallel", "arbitrary")),
    )(cgf, cgb, h0f, c0f, h0b, c0b, bhf, bhb, biasf, biasb)


def _lstm_body(pf_ref, pb_ref, pfr_ref, pbr_ref,
               h0f_ref, c0f_ref, h0b_ref, c0b_ref,
               a1f_ref, a2f_ref, bhf_ref, a1b_ref, a2b_ref, bhb_ref,
               bf_ref, bb_ref,
               of_ref, ob_ref, hfo_ref, cfo_ref, hbo_ref, cbo_ref,
               hf_s, cf_s, hb_s, cb_s, gxf_s, gxb_s):
    j = pl.program_id(1)

    @pl.when(j == 0)
    def _():
        hf_s[...] = h0f_ref[...]
        cf_s[...] = c0f_ref[...]
        hb_s[...] = h0b_ref[...]
        cb_s[...] = c0b_ref[...]

    zf = pf_ref[...].reshape(TC * BK, H)
    zb = pb_ref[...].reshape(TC * BK, H)
    gxf = (jnp.dot(zf, a1f_ref[...], preferred_element_type=_f32)
           + jnp.dot(zb, a2f_ref[...], preferred_element_type=_f32)
           + bf_ref[...])
    gxf_s[...] = gxf.reshape(TC, BK, GP)
    zfr = pfr_ref[...].reshape(TC * BK, H)
    zbr = pbr_ref[...].reshape(TC * BK, H)
    gxb = (jnp.dot(zfr, a1b_ref[...], preferred_element_type=_f32)
           + jnp.dot(zbr, a2b_ref[...], preferred_element_type=_f32)
           + bb_ref[...])
    gxb_s[...] = gxb.reshape(TC, BK, GP)

    def step(tt, carry):
        hf, cf, hb, cb = carry
        rt = TC - 1 - tt
        ghf = gxf_s[tt] + jnp.dot(hf, bhf_ref[...],
                                  preferred_element_type=_f32)
        h2f, c2f = _gates([ghf[:, k * SP:k * SP + H] for k in range(4)], cf)
        vf = (j * TC + tt) < C
        hf = jnp.where(vf, h2f, hf)
        cf = jnp.where(vf, c2f, cf)
        of_ref[tt] = hf.astype(jnp.bfloat16)

        ghb = gxb_s[rt] + jnp.dot(hb, bhb_ref[...],
                                  preferred_element_type=_f32)
        h2b, c2b = _gates([ghb[:, k * SP:k * SP + H] for k in range(4)], cb)
        vb = (j * TC + tt) >= (TP - C)
        hb = jnp.where(vb, h2b, hb)
        cb = jnp.where(vb, c2b, cb)
        ob_ref[rt] = hb.astype(jnp.bfloat16)
        return hf, cf, hb, cb

    carry = (hf_s[...], cf_s[...], hb_s[...], cb_s[...])
    for tt in range(TC):
        carry = step(tt, carry)
    hf, cf, hb, cb = carry
    hf_s[...] = hf
    cf_s[...] = cf
    hb_s[...] = hb
    cb_s[...] = cb
    hfo_ref[...] = hf
    cfo_ref[...] = cf
    hbo_ref[...] = hb
    cbo_ref[...] = cb


def _lstm(pf, pb, h0f, c0f, h0b, c0b, a1f, a2f, bhf, a1b, a2b, bhb,
          biasf, biasb):
    state = jax.ShapeDtypeStruct((B, H), _f32)
    seq = jax.ShapeDtypeStruct((TP, B, H), jnp.bfloat16)
    wspec = pl.BlockSpec((H, GP), lambda i, j: (0, 0))
    sspec = pl.BlockSpec((BK, H), lambda i, j: (i, 0))
    fwd = pl.BlockSpec((TC, BK, H), lambda i, j: (j, i, 0))
    rev = pl.BlockSpec((TC, BK, H), lambda i, j: (NCH - 1 - j, i, 0))
    return pl.pallas_call(
        _lstm_body,
        grid=(2, NCH),
        in_specs=[
            fwd, fwd, rev, rev,
            sspec, sspec, sspec, sspec,
            wspec, wspec, wspec, wspec, wspec, wspec,
            pl.BlockSpec((1, GP), lambda i, j: (0, 0)),
            pl.BlockSpec((1, GP), lambda i, j: (0, 0)),
        ],
        out_specs=[fwd, rev, sspec, sspec, sspec, sspec],
        out_shape=[seq, seq, state, state, state, state],
        scratch_shapes=([pltpu.VMEM((BK, H), _f32) for _ in range(4)]
                        + [pltpu.VMEM((TC, BK, GP), _f32) for _ in range(2)]),
        compiler_params=_cparams(("parallel", "arbitrary")),
    )(pf, pb, pf, pb, h0f, c0f, h0b, c0b, a1f, a2f, bhf, a1b, a2b, bhb,
      biasf, biasb)


def _k4_body(f_ref, b_ref, wf_ref, wb_ref, bias_ref, y_ref):
    y = (jnp.dot(f_ref[...].reshape(TC * BK, H), wf_ref[...],
                 preferred_element_type=_f32)
         + jnp.dot(b_ref[...].reshape(TC * BK, H), wb_ref[...],
                   preferred_element_type=_f32)
         + bias_ref[...])
    y_ref[...] = y.reshape(TC, BK, 2)


def _k4(of, ob, wf, wb, bias):
    fwd = pl.BlockSpec((TC, BK, H), lambda i, j: (j, i, 0))
    return pl.pallas_call(
        _k4_body,
        grid=(2, NCH),
        in_specs=[
            fwd, fwd,
            pl.BlockSpec((H, 2), lambda i, j: (0, 0)),
            pl.BlockSpec((H, 2), lambda i, j: (0, 0)),
            pl.BlockSpec((1, 2), lambda i, j: (0, 0)),
        ],
        out_specs=pl.BlockSpec((TC, BK, 2), lambda i, j: (j, i, 0)),
        out_shape=jax.ShapeDtypeStruct((TP, B, 2), _f32),
        compiler_params=_cparams(("parallel", "arbitrary")),
    )(of, ob, wf, wb, bias)


def _pad_stripes(wt):
    parts = []
    for k in range(4):
        p = wt[:, k * H:(k + 1) * H]
        parts.append(jnp.pad(p, ((0, 0), (0, SP - H))))
    return jnp.concatenate(parts, axis=1)


def kernel(tok_feats, offset_mapping, h0, c0, w_lin, b_lin, w_ih0_f, w_ih0_b,
           w_ih_f, w_ih_b, w_hh_f, w_hh_b, b_f, b_b, w1, b1, w2, b2):
    wlinT = w_lin.T
    wcatT = jnp.concatenate([w_ih0_f, w_ih0_b], axis=0).T
    wcomb = _k1(wlinT, wcatT)
    blin = jnp.concatenate([b_lin @ w_ih0_f.T, b_lin @ w_ih0_b.T])[None, :]
    starts = offset_mapping[..., 0]
    ends = offset_mapping[..., 1]
    cgf, cgb = _k2(tok_feats, wcomb, blin, starts, ends)

    bias_f0 = _pad_stripes((b_f[0])[None, :])
    bias_b0 = _pad_stripes((b_b[0])[None, :])

    h0f = [h0[2 * l] for l in range(L)]
    h0b = [h0[2 * l + 1] for l in range(L)]
    c0f = [c0[2 * l] for l in range(L)]
    c0b = [c0[2 * l + 1] for l in range(L)]

    bhf = [_pad_stripes(w_hh_f[l].T) for l in range(L)]
    bhb = [_pad_stripes(w_hh_b[l].T) for l in range(L)]

    of, ob, hf, cf, hb, cb = _lstm0(cgf, cgb, h0f[0], c0f[0], h0b[0], c0b[0],
                                    bhf[0], bhb[0], bias_f0, bias_b0)
    hs = [hf, hb]
    cs = [cf, cb]
    for l in range(1, L):
        wtf = w_ih_f[l - 1].T
        wtb = w_ih_b[l - 1].T
        a1f = _pad_stripes(wtf[:H]).astype(jnp.bfloat16)
        a2f = _pad_stripes(wtf[H:]).astype(jnp.bfloat16)
        a1b = _pad_stripes(wtb[:H]).astype(jnp.bfloat16)
        a2b = _pad_stripes(wtb[H:]).astype(jnp.bfloat16)
        biasf = _pad_stripes((b_f[l])[None, :])
        biasb = _pad_stripes((b_b[l])[None, :])
        of, ob, hf, cf, hb, cb = _lstm(of, ob, h0f[l], c0f[l], h0b[l], c0b[l],
                                       a1f, a2f, bhf[l], a1b, a2b, bhb[l],
                                       biasf, biasb)
        hs += [hf, hb]
        cs += [cf, cb]

    w12 = w2 @ w1
    b12 = b2 + b1 @ w2.T
    w12t = w12.T
    y = _k4(of, ob, w12t[:H].astype(jnp.bfloat16),
            w12t[H:].astype(jnp.bfloat16), b12[None, :])

    yt = jnp.transpose(y[:C], (1, 0, 2))
    hn = jnp.stack(hs)
    cn = jnp.stack(cs)
    return yt[..., :1], yt[..., 1:], hn, cn
